# baseline (speedup 1.0000x reference)
import jax
import jax.numpy as jnp
from jax.experimental import pallas as pl
from jax.experimental.pallas import tpu as pltpu

_LANES = 128
_R = 4096
_NBUF = 8


def _body(vecs, ids, ones, queue, qids, valid8,
          oq, oids, oval, qbuf, idbuf, vbuf, sin, sout):
    batch = vecs.shape[0]
    k = queue.shape[0]
    nc = k // _R
    nv = batch // _R
    rr = _R // _LANES

    def srcs(c):
        off = c * _R
        off2 = c * rr
        if c < nv:
            return (vecs.at[pl.ds(off, _R)], ids.at[pl.ds(off2, rr)],
                    ones.at[pl.ds(off2, rr)])
        return (queue.at[pl.ds(off, _R)], qids.at[pl.ds(off2, rr)],
                valid8.at[pl.ds(off2, rr)])

    def in_copies(c):
        b = c % _NBUF
        sq, si, sv = srcs(c)
        return (pltpu.make_async_copy(sq, qbuf.at[b], sin.at[0, b]),
                pltpu.make_async_copy(si, idbuf.at[b], sin.at[1, b]),
                pltpu.make_async_copy(sv, vbuf.at[b], sin.at[2, b]))

    def out_copies(c):
        b = c % _NBUF
        off = c * _R
        off2 = c * rr
        return (pltpu.make_async_copy(qbuf.at[b], oq.at[pl.ds(off, _R)], sout.at[0, b]),
                pltpu.make_async_copy(idbuf.at[b], oids.at[pl.ds(off2, rr)], sout.at[1, b]),
                pltpu.make_async_copy(vbuf.at[b], oval.at[pl.ds(off2, rr)], sout.at[2, b]))

    for c in range(nc):
        if c >= _NBUF:
            for cp in out_copies(c - _NBUF):
                cp.wait()
        for cp in in_copies(c):
            cp.start()
        if c >= 1:
            for cp in in_copies(c - 1):
                cp.wait()
            for cp in out_copies(c - 1):
                cp.start()
    for cp in in_copies(nc - 1):
        cp.wait()
    for cp in out_copies(nc - 1):
        cp.start()
    for c in range(nc - _NBUF, nc):
        for cp in out_copies(c):
            cp.wait()


def kernel(vecs, ids, queue, queue_ids, valid):
    batch, dim = vecs.shape
    k = queue.shape[0]
    rr = _R // _LANES
    ids2d = ids.reshape(batch // _LANES, _LANES)
    ones2d = jnp.ones((batch // _LANES, _LANES), dtype=jnp.int8)
    qids2d = queue_ids.reshape(k // _LANES, _LANES)
    valid8 = valid.astype(jnp.int8).reshape(k // _LANES, _LANES)

    hbm = pl.BlockSpec(memory_space=pltpu.MemorySpace.HBM)
    oq, oids2d, oval8 = pl.pallas_call(
        _body,
        in_specs=[hbm] * 6,
        out_specs=[hbm] * 3,
        out_shape=[
            jax.ShapeDtypeStruct((k, dim), queue.dtype),
            jax.ShapeDtypeStruct((k // _LANES, _LANES), queue_ids.dtype),
            jax.ShapeDtypeStruct((k // _LANES, _LANES), jnp.int8),
        ],
        scratch_shapes=[
            pltpu.VMEM((_NBUF, _R, dim), queue.dtype),
            pltpu.VMEM((_NBUF, rr, _LANES), queue_ids.dtype),
            pltpu.VMEM((_NBUF, rr, _LANES), jnp.int8),
            pltpu.SemaphoreType.DMA((3, _NBUF)),
            pltpu.SemaphoreType.DMA((3, _NBUF)),
        ],
    )(vecs, ids2d, ones2d, queue, qids2d, valid8)
    return (oq, oids2d.reshape(k), oval8.reshape(k).astype(jnp.bool_))


# speedup vs baseline: 1.0949x; 1.0949x over previous
import jax
import jax.numpy as jnp
from jax.experimental import pallas as pl
from jax.experimental.pallas import tpu as pltpu

_LANES = 128
_R = 16384
_NBUF = 4


def _body(vecs, ids, ones, queue, qids, valid8,
          oq, oids, oval, qbuf, idbuf, vbuf, sin, sout):
    batch = vecs.shape[0]
    k = queue.shape[0]
    nc = k // _R
    nv = batch // _R
    rr = _R // _LANES

    def srcs(c):
        off = c * _R
        off2 = c * rr
        if c < nv:
            return (vecs.at[pl.ds(off, _R)], ids.at[pl.ds(off2, rr)],
                    ones.at[pl.ds(off2, rr)])
        return (queue.at[pl.ds(off, _R)], qids.at[pl.ds(off2, rr)],
                valid8.at[pl.ds(off2, rr)])

    def in_copies(c):
        b = c % _NBUF
        sq, si, sv = srcs(c)
        return (pltpu.make_async_copy(sq, qbuf.at[b], sin.at[0, b]),
                pltpu.make_async_copy(si, idbuf.at[b], sin.at[1, b]),
                pltpu.make_async_copy(sv, vbuf.at[b], sin.at[2, b]))

    def out_copies(c):
        b = c % _NBUF
        off = c * _R
        off2 = c * rr
        return (pltpu.make_async_copy(qbuf.at[b], oq.at[pl.ds(off, _R)], sout.at[0, b]),
                pltpu.make_async_copy(idbuf.at[b], oids.at[pl.ds(off2, rr)], sout.at[1, b]),
                pltpu.make_async_copy(vbuf.at[b], oval.at[pl.ds(off2, rr)], sout.at[2, b]))

    for c in range(nc):
        if c >= _NBUF:
            for cp in out_copies(c - _NBUF):
                cp.wait()
        for cp in in_copies(c):
            cp.start()
        if c >= 1:
            for cp in in_copies(c - 1):
                cp.wait()
            for cp in out_copies(c - 1):
                cp.start()
    for cp in in_copies(nc - 1):
        cp.wait()
    for cp in out_copies(nc - 1):
        cp.start()
    for c in range(nc - _NBUF, nc):
        for cp in out_copies(c):
            cp.wait()


def kernel(vecs, ids, queue, queue_ids, valid):
    batch, dim = vecs.shape
    k = queue.shape[0]
    rr = _R // _LANES
    ids2d = ids.reshape(batch // _LANES, _LANES)
    ones2d = jnp.ones((batch // _LANES, _LANES), dtype=jnp.int8)
    qids2d = queue_ids.reshape(k // _LANES, _LANES)
    valid8 = valid.astype(jnp.int8).reshape(k // _LANES, _LANES)

    hbm = pl.BlockSpec(memory_space=pltpu.MemorySpace.HBM)
    oq, oids2d, oval8 = pl.pallas_call(
        _body,
        in_specs=[hbm] * 6,
        out_specs=[hbm] * 3,
        out_shape=[
            jax.ShapeDtypeStruct((k, dim), queue.dtype),
            jax.ShapeDtypeStruct((k // _LANES, _LANES), queue_ids.dtype),
            jax.ShapeDtypeStruct((k // _LANES, _LANES), jnp.int8),
        ],
        scratch_shapes=[
            pltpu.VMEM((_NBUF, _R, dim), queue.dtype),
            pltpu.VMEM((_NBUF, rr, _LANES), queue_ids.dtype),
            pltpu.VMEM((_NBUF, rr, _LANES), jnp.int8),
            pltpu.SemaphoreType.DMA((3, _NBUF)),
            pltpu.SemaphoreType.DMA((3, _NBUF)),
        ],
    )(vecs, ids2d, ones2d, queue, qids2d, valid8)
    return (oq, oids2d.reshape(k), oval8.reshape(k).astype(jnp.bool_))


# speedup vs baseline: 1.1057x; 1.0099x over previous
import jax
import jax.numpy as jnp
from jax.experimental import pallas as pl
from jax.experimental.pallas import tpu as pltpu

_LANES = 128
_R = 32768
_NBUF = 3


def _body(vecs, ids, ones, queue, qids, valid8,
          oq, oids, oval, qbuf, idbuf, vbuf, sin, sout):
    batch = vecs.shape[0]
    k = queue.shape[0]
    nc = k // _R
    rr = _R // _LANES
    br = batch // _LANES

    def in_copies(c):
        b = c % _NBUF
        lo, hi = c * _R, (c + 1) * _R
        cps = []
        slot = 0
        for seg_lo, seg_hi, q_src, i_src, v_src in (
                (lo, min(hi, batch), vecs, ids, ones),
                (max(lo, batch), hi, queue, qids, valid8)):
            if seg_lo >= seg_hi:
                continue
            n = seg_hi - seg_lo
            n2 = n // _LANES
            d0 = seg_lo - lo
            d2 = d0 // _LANES
            s2 = seg_lo // _LANES
            cps += [
                pltpu.make_async_copy(q_src.at[pl.ds(seg_lo, n)],
                                      qbuf.at[b, pl.ds(d0, n)], sin.at[slot + 0, b]),
                pltpu.make_async_copy(i_src.at[pl.ds(s2, n2)],
                                      idbuf.at[b, pl.ds(d2, n2)], sin.at[slot + 1, b]),
                pltpu.make_async_copy(v_src.at[pl.ds(s2, n2)],
                                      vbuf.at[b, pl.ds(d2, n2)], sin.at[slot + 2, b]),
            ]
            slot += 3
        return cps

    def out_copies(c):
        b = c % _NBUF
        off = c * _R
        off2 = c * rr
        return (pltpu.make_async_copy(qbuf.at[b], oq.at[pl.ds(off, _R)], sout.at[0, b]),
                pltpu.make_async_copy(idbuf.at[b], oids.at[pl.ds(off2, rr)], sout.at[1, b]),
                pltpu.make_async_copy(vbuf.at[b], oval.at[pl.ds(off2, rr)], sout.at[2, b]))

    for c in range(nc):
        if c >= _NBUF:
            for cp in out_copies(c - _NBUF):
                cp.wait()
        for cp in in_copies(c):
            cp.start()
        if c >= 1:
            for cp in in_copies(c - 1):
                cp.wait()
            for cp in out_copies(c - 1):
                cp.start()
    for cp in in_copies(nc - 1):
        cp.wait()
    for cp in out_copies(nc - 1):
        cp.start()
    for c in range(nc - _NBUF, nc):
        for cp in out_copies(c):
            cp.wait()


def kernel(vecs, ids, queue, queue_ids, valid):
    batch, dim = vecs.shape
    k = queue.shape[0]
    rr = _R // _LANES
    ids2d = ids.reshape(batch // _LANES, _LANES)
    ones2d = jnp.ones((batch // _LANES, _LANES), dtype=jnp.int8)
    qids2d = queue_ids.reshape(k // _LANES, _LANES)
    valid8 = valid.astype(jnp.int8).reshape(k // _LANES, _LANES)

    hbm = pl.BlockSpec(memory_space=pltpu.MemorySpace.HBM)
    oq, oids2d, oval8 = pl.pallas_call(
        _body,
        in_specs=[hbm] * 6,
        out_specs=[hbm] * 3,
        out_shape=[
            jax.ShapeDtypeStruct((k, dim), queue.dtype),
            jax.ShapeDtypeStruct((k // _LANES, _LANES), queue_ids.dtype),
            jax.ShapeDtypeStruct((k // _LANES, _LANES), jnp.int8),
        ],
        scratch_shapes=[
            pltpu.VMEM((_NBUF, _R, dim), queue.dtype),
            pltpu.VMEM((_NBUF, rr, _LANES), queue_ids.dtype),
            pltpu.VMEM((_NBUF, rr, _LANES), jnp.int8),
            pltpu.SemaphoreType.DMA((6, _NBUF)),
            pltpu.SemaphoreType.DMA((3, _NBUF)),
        ],
    )(vecs, ids2d, ones2d, queue, qids2d, valid8)
    return (oq, oids2d.reshape(k), oval8.reshape(k).astype(jnp.bool_))
